# Initial kernel scaffold; baseline (speedup 1.0000x reference)
import jax
import jax.numpy as jnp
from jax.experimental import pallas as pl


def kernel(FM, rois):
    raise NotImplementedError("write your pallas kernel here")


# speedup vs baseline: 41.8452x; 41.8452x over previous
import functools

import jax
import jax.numpy as jnp
from jax import lax
from jax.experimental import pallas as pl
from jax.experimental.pallas import tpu as pltpu
from jax.experimental.pallas import tpu_sc as plsc

H = 56
W = 56
C = 256
NROI = 512
P = 7
NBIN = P * P
NTAB = 16
HW = H * W

NC = 2
NS = 16
NWORK = NC * NS
RPW = NROI // NWORK
SEG = 56
ROISEG = 4 * SEG


def _tables_body(fmt_ref, out_ref):
    kh = pl.program_id(0)
    X = fmt_ref[...]
    for k in range(3):
        s = 1 << k
        sh = jnp.concatenate(
            [X[s:], jnp.broadcast_to(X[-1:], (s, W, C))], axis=0)
        X = jnp.where(kh > k, jnp.maximum(X, sh), X)
    Y = X
    for kw in range(4):
        if kw > 0:
            s = 1 << (kw - 1)
            sh = jnp.concatenate(
                [Y[:, s:], jnp.broadcast_to(Y[:, -1:], (H, s, C))], axis=1)
            Y = jnp.maximum(Y, sh)
        out_ref[0, kw] = Y


def _build_tables(FMt):
    return pl.pallas_call(
        _tables_body,
        grid=(4,),
        in_specs=[pl.BlockSpec((H, W, C), lambda g: (0, 0, 0))],
        out_specs=pl.BlockSpec((1, 4, H, W, C), lambda g: (g, 0, 0, 0, 0)),
        out_shape=jax.ShapeDtypeStruct((4, 4, H, W, C), jnp.float32),
    )(FMt)


def _rint_nonneg(x):
    fl = x.astype(jnp.int32)
    fr = x - fl.astype(jnp.float32)
    odd = (fl & 1) == 1
    up = (fr > 0.5) | ((fr == 0.5) & odd)
    return fl + up.astype(jnp.int32)


def _sc_body(tabs, roist, out, rv, idxb, rows, outT, sem):
    cid = lax.axis_index("c")
    sid = lax.axis_index("s")
    wid = sid * NC + cid
    base = wid * RPW

    for d in range(4):
        pltpu.sync_copy(roist.at[d, pl.ds(base * 1, RPW)], rv.at[d])

    lane = jnp.arange(RPW, dtype=jnp.int32)
    zero = jnp.zeros((RPW,), jnp.int32)

    for q in range(4):
        for k in range(NBIN, SEG):
            plsc.store_scatter(idxb, [lane * ROISEG + (q * SEG + k)], zero)

    fi = rv[0]
    fj = rv[1]
    fh = rv[2]
    fw = rv[3]
    y0 = jnp.clip(_rint_nonneg(fi * float(H)), 0, H - 1)
    x0 = jnp.clip(_rint_nonneg(fj * float(W)), 0, W - 1)
    rh = jnp.minimum(jnp.maximum(_rint_nonneg(fh * float(H)), 1), H - y0)
    rw = jnp.minimum(jnp.maximum(_rint_nonneg(fw * float(W)), 1), W - x0)

    def edges(p, v0, rv_):
        s = v0 + (p * rv_) // P
        e = v0 + ((p + 1) * rv_ + (P - 1)) // P
        e = jnp.maximum(e, s + 1)
        d = e - s
        pw = jnp.where(d >= 8, 8, jnp.where(d >= 4, 4, jnp.where(d >= 2, 2, 1)))
        kk = (
            (d >= 2).astype(jnp.int32)
            + (d >= 4).astype(jnp.int32)
            + (d >= 8).astype(jnp.int32)
        )
        return s, e - pw, kk

    hA = []
    wA = []
    for p in range(P):
        hA.append(edges(p, y0, rh))
        wA.append(edges(p, x0, rw))

    for py in range(P):
        r0, r1, kh = hA[py]
        for px in range(P):
            c0, c1, kw = wA[px]
            tb = (kh * 4 + kw) * HW
            b = py * P + px
            for q, (rr, cc) in enumerate(((r0, c0), (r0, c1), (r1, c0), (r1, c1))):
                plsc.store_scatter(
                    idxb, [lane * ROISEG + (q * SEG + b)], tb + rr * W + cc)

    def per_roi(r, carry):
        copies = []
        for q in range(4):
            copies.append(pltpu.async_copy(
                tabs.at[idxb.at[pl.ds(r * ROISEG + q * SEG, SEG)]],
                rows.at[q], sem))
        for cpy in copies:
            cpy.wait()

        def per_bin(b, carry2):
            for v in range(C // 16):
                m = jnp.maximum(
                    jnp.maximum(rows[0, b, pl.ds(16 * v, 16)],
                                rows[1, b, pl.ds(16 * v, 16)]),
                    jnp.maximum(rows[2, b, pl.ds(16 * v, 16)],
                                rows[3, b, pl.ds(16 * v, 16)]))
                cidx = (16 * v + jnp.arange(16, dtype=jnp.int32)) * NBIN + b
                plsc.store_scatter(outT, [cidx], m)
            return carry2

        lax.fori_loop(0, NBIN, per_bin, 0)
        pltpu.sync_copy(outT, out.at[base + r])
        return carry

    lax.fori_loop(0, RPW, per_roi, 0)


def _sc_pool(tabs, roist):
    mesh = plsc.VectorSubcoreMesh(core_axis_name="c", subcore_axis_name="s")
    return pl.kernel(
        _sc_body,
        mesh=mesh,
        compiler_params=pltpu.CompilerParams(needs_layout_passes=False),
        out_type=jax.ShapeDtypeStruct((NROI, C * NBIN), jnp.float32),
        scratch_types=[
            pltpu.VMEM((4, RPW), jnp.float32),
            pltpu.VMEM((RPW * ROISEG,), jnp.int32),
            pltpu.VMEM((4, SEG, C), jnp.float32),
            pltpu.VMEM((C * NBIN,), jnp.float32),
            pltpu.SemaphoreType.DMA,
        ],
    )(tabs, roist)


@jax.jit
def kernel(FM, rois):
    FMt = jnp.transpose(FM, (1, 2, 0))
    tabs = _build_tables(FMt).reshape(NTAB * HW, C)
    roist = jnp.transpose(rois, (1, 0))
    out = _sc_pool(tabs, roist)
    return out.reshape(NROI, C, P, P)


# speedup vs baseline: 71.0853x; 1.6988x over previous
import functools

import jax
import jax.numpy as jnp
from jax import lax
from jax.experimental import pallas as pl
from jax.experimental.pallas import tpu as pltpu
from jax.experimental.pallas import tpu_sc as plsc

H = 56
W = 56
C = 256
NROI = 512
P = 7
NBIN = P * P
NTAB = 16
HW = H * W

NC = 2
NS = 16
NWORK = NC * NS
RPW = NROI // NWORK
ROISEG = 208
GROWS = 104


def _tables_body(fmt_ref, out_ref):
    kh = pl.program_id(0)
    X = fmt_ref[...]
    for k in range(3):
        s = 1 << k
        sh = jnp.concatenate(
            [X[s:], jnp.broadcast_to(X[-1:], (s, W, C))], axis=0)
        X = jnp.where(kh > k, jnp.maximum(X, sh), X)
    Y = X
    for kw in range(4):
        if kw > 0:
            s = 1 << (kw - 1)
            sh = jnp.concatenate(
                [Y[:, s:], jnp.broadcast_to(Y[:, -1:], (H, s, C))], axis=1)
            Y = jnp.maximum(Y, sh)
        out_ref[0, kw] = Y


def _build_tables(FMt):
    return pl.pallas_call(
        _tables_body,
        grid=(4,),
        in_specs=[pl.BlockSpec((H, W, C), lambda g: (0, 0, 0))],
        out_specs=pl.BlockSpec((1, 4, H, W, C), lambda g: (g, 0, 0, 0, 0)),
        out_shape=jax.ShapeDtypeStruct((4, 4, H, W, C), jnp.float32),
    )(FMt)


def _rint_nonneg(x):
    fl = x.astype(jnp.int32)
    fr = x - fl.astype(jnp.float32)
    odd = (fl & 1) == 1
    up = (fr > 0.5) | ((fr == 0.5) & odd)
    return fl + up.astype(jnp.int32)


def _sc_body(tabs, roist, out, rv, idxb, rows, outT, semA, semB):
    cid = lax.axis_index("c")
    sid = lax.axis_index("s")
    wid = sid * NC + cid
    base = wid * RPW

    for d in range(4):
        pltpu.sync_copy(roist.at[d, pl.ds(base * 1, RPW)], rv.at[d])

    lane = jnp.arange(RPW, dtype=jnp.int32)
    zero = jnp.zeros((RPW,), jnp.int32)

    for k in range(4 * NBIN, ROISEG):
        plsc.store_scatter(idxb, [lane * ROISEG + k], zero)

    fi = rv[0]
    fj = rv[1]
    fh = rv[2]
    fw = rv[3]
    y0 = jnp.clip(_rint_nonneg(fi * float(H)), 0, H - 1)
    x0 = jnp.clip(_rint_nonneg(fj * float(W)), 0, W - 1)
    rh = jnp.minimum(jnp.maximum(_rint_nonneg(fh * float(H)), 1), H - y0)
    rw = jnp.minimum(jnp.maximum(_rint_nonneg(fw * float(W)), 1), W - x0)

    def edges(p, v0, rv_):
        s = v0 + (p * rv_) // P
        e = v0 + ((p + 1) * rv_ + (P - 1)) // P
        e = jnp.maximum(e, s + 1)
        d = e - s
        pw = jnp.where(d >= 8, 8, jnp.where(d >= 4, 4, jnp.where(d >= 2, 2, 1)))
        kk = (
            (d >= 2).astype(jnp.int32)
            + (d >= 4).astype(jnp.int32)
            + (d >= 8).astype(jnp.int32)
        )
        return s, e - pw, kk

    hA = []
    wA = []
    for p in range(P):
        hA.append(edges(p, y0, rh))
        wA.append(edges(p, x0, rw))

    for py in range(P):
        r0, r1, kh = hA[py]
        for px in range(P):
            c0, c1, kw = wA[px]
            tb = (kh * 4 + kw) * HW
            b = py * P + px
            for q, (rr, cc) in enumerate(((r0, c0), (r0, c1), (r1, c0), (r1, c1))):
                plsc.store_scatter(
                    idxb, [lane * ROISEG + (q * NBIN + b)], tb + rr * W + cc)

    sems = (semA, semB)
    ivec = jnp.arange(16, dtype=jnp.int32) * NBIN

    def issue(r):
        buf = r % 2
        return [
            pltpu.async_copy(
                tabs.at[idxb.at[pl.ds(r * ROISEG + g * GROWS, GROWS)]],
                rows.at[buf, pl.ds(g * GROWS, GROWS)], sems[buf])
            for g in range(2)
        ]

    pending = {0: issue(0)}
    for r in range(RPW):
        for cpy in pending.pop(r):
            cpy.wait()
        if r + 1 < RPW:
            pending[r + 1] = issue(r + 1)
        buf = r % 2

        def per_bin(b, carry2, buf=buf):
            for v in range(C // 16):
                m = jnp.maximum(
                    jnp.maximum(rows[buf, b, pl.ds(16 * v, 16)],
                                rows[buf, NBIN + b, pl.ds(16 * v, 16)]),
                    jnp.maximum(rows[buf, 2 * NBIN + b, pl.ds(16 * v, 16)],
                                rows[buf, 3 * NBIN + b, pl.ds(16 * v, 16)]))
                plsc.store_scatter(outT, [ivec + (16 * NBIN * v + b)], m)
            return carry2

        lax.fori_loop(0, NBIN, per_bin, 0)
        pltpu.sync_copy(outT, out.at[base + r])


def _sc_pool(tabs, roist):
    mesh = plsc.VectorSubcoreMesh(core_axis_name="c", subcore_axis_name="s")
    return pl.kernel(
        _sc_body,
        mesh=mesh,
        compiler_params=pltpu.CompilerParams(needs_layout_passes=False),
        out_type=jax.ShapeDtypeStruct((NROI, C * NBIN), jnp.float32),
        scratch_types=[
            pltpu.VMEM((4, RPW), jnp.float32),
            pltpu.VMEM((RPW * ROISEG,), jnp.int32),
            pltpu.VMEM((2, ROISEG, C), jnp.float32),
            pltpu.VMEM((C * NBIN,), jnp.float32),
            pltpu.SemaphoreType.DMA,
            pltpu.SemaphoreType.DMA,
        ],
    )(tabs, roist)


@jax.jit
def kernel(FM, rois):
    FMt = jnp.transpose(FM, (1, 2, 0))
    tabs = _build_tables(FMt).reshape(NTAB * HW, C)
    roist = jnp.transpose(rois, (1, 0))
    out = _sc_pool(tabs, roist)
    return out.reshape(NROI, C, P, P)


# speedup vs baseline: 71.2362x; 1.0021x over previous
import functools

import jax
import jax.numpy as jnp
from jax import lax
from jax.experimental import pallas as pl
from jax.experimental.pallas import tpu as pltpu
from jax.experimental.pallas import tpu_sc as plsc

H = 56
W = 56
C = 256
NROI = 512
P = 7
NBIN = P * P
NTAB = 16
HW = H * W

NC = 2
NS = 16
NWORK = NC * NS
RPW = NROI // NWORK
ROISEG = 208
GROWS = 104


def _tables_body(fmt_ref, out_ref):
    kh = pl.program_id(0)
    X = fmt_ref[...]
    for k in range(3):
        s = 1 << k
        sh = jnp.concatenate(
            [X[s:], jnp.broadcast_to(X[-1:], (s, W, C))], axis=0)
        X = jnp.where(kh > k, jnp.maximum(X, sh), X)
    Y = X
    for kw in range(4):
        if kw > 0:
            s = 1 << (kw - 1)
            sh = jnp.concatenate(
                [Y[:, s:], jnp.broadcast_to(Y[:, -1:], (H, s, C))], axis=1)
            Y = jnp.maximum(Y, sh)
        out_ref[0, kw] = Y


def _build_tables(FMt):
    return pl.pallas_call(
        _tables_body,
        grid=(4,),
        in_specs=[pl.BlockSpec((H, W, C), lambda g: (0, 0, 0))],
        out_specs=pl.BlockSpec((1, 4, H, W, C), lambda g: (g, 0, 0, 0, 0)),
        out_shape=jax.ShapeDtypeStruct((4, 4, H, W, C), jnp.float32),
    )(FMt)


def _rint_nonneg(x):
    fl = x.astype(jnp.int32)
    fr = x - fl.astype(jnp.float32)
    odd = (fl & 1) == 1
    up = (fr > 0.5) | ((fr == 0.5) & odd)
    return fl + up.astype(jnp.int32)


def _sc_body(tabs, roist, out, rv, idxb, rows, outT, semA, semB):
    cid = lax.axis_index("c")
    sid = lax.axis_index("s")
    wid = sid * NC + cid
    base = wid * RPW

    for d in range(4):
        pltpu.sync_copy(roist.at[d, pl.ds(base * 1, RPW)], rv.at[d])

    lane = jnp.arange(RPW, dtype=jnp.int32)
    zero = jnp.zeros((RPW,), jnp.int32)

    for k in range(4 * NBIN, ROISEG):
        plsc.store_scatter(idxb, [lane * ROISEG + k], zero)

    fi = rv[0]
    fj = rv[1]
    fh = rv[2]
    fw = rv[3]
    y0 = jnp.clip(_rint_nonneg(fi * float(H)), 0, H - 1)
    x0 = jnp.clip(_rint_nonneg(fj * float(W)), 0, W - 1)
    rh = jnp.minimum(jnp.maximum(_rint_nonneg(fh * float(H)), 1), H - y0)
    rw = jnp.minimum(jnp.maximum(_rint_nonneg(fw * float(W)), 1), W - x0)

    def edges(p, v0, rv_):
        s = v0 + (p * rv_) // P
        e = v0 + ((p + 1) * rv_ + (P - 1)) // P
        e = jnp.maximum(e, s + 1)
        d = e - s
        pw = jnp.where(d >= 8, 8, jnp.where(d >= 4, 4, jnp.where(d >= 2, 2, 1)))
        kk = (
            (d >= 2).astype(jnp.int32)
            + (d >= 4).astype(jnp.int32)
            + (d >= 8).astype(jnp.int32)
        )
        return s, e - pw, kk

    hA = []
    wA = []
    for p in range(P):
        hA.append(edges(p, y0, rh))
        wA.append(edges(p, x0, rw))

    for py in range(P):
        r0, r1, kh = hA[py]
        for px in range(P):
            c0, c1, kw = wA[px]
            tb = (kh * 4 + kw) * HW
            b = py * P + px
            for q, (rr, cc) in enumerate(((r0, c0), (r0, c1), (r1, c0), (r1, c1))):
                plsc.store_scatter(
                    idxb, [lane * ROISEG + (q * NBIN + b)], tb + rr * W + cc)

    sems = (semA, semB)
    ivec = jnp.arange(16, dtype=jnp.int32) * NBIN

    def issue(r, buf):
        for g in range(2):
            pltpu.async_copy(
                tabs.at[idxb.at[pl.ds(r * ROISEG + g * GROWS, GROWS)]],
                rows.at[buf, pl.ds(g * GROWS, GROWS)], sems[buf])

    def drain(buf):
        for g in range(2):
            pltpu.make_async_copy(
                tabs.at[pl.ds(0, GROWS)],
                rows.at[buf, pl.ds(g * GROWS, GROWS)], sems[buf]).wait()

    def compute(r, buf):
        def per_row(py, carry):
            b0 = py * P
            for px in range(P):
                bvec = ivec + (b0 + px)
                for v in range(C // 16):
                    m = jnp.maximum(
                        jnp.maximum(
                            rows[buf, b0 + px, pl.ds(16 * v, 16)],
                            rows[buf, NBIN + b0 + px, pl.ds(16 * v, 16)]),
                        jnp.maximum(
                            rows[buf, 2 * NBIN + b0 + px, pl.ds(16 * v, 16)],
                            rows[buf, 3 * NBIN + b0 + px, pl.ds(16 * v, 16)]))
                    plsc.store_scatter(outT, [bvec + (16 * NBIN * v)], m)
            return carry

        lax.fori_loop(0, P, per_row, 0)
        pltpu.sync_copy(outT, out.at[base + r])

    issue(0, 0)
    issue(1, 1)

    def pair(g, carry):
        ra = 2 * g
        drain(0)
        compute(ra, 0)

        @pl.when(ra + 2 < RPW)
        def _():
            issue(ra + 2, 0)

        drain(1)
        compute(ra + 1, 1)

        @pl.when(ra + 3 < RPW)
        def _():
            issue(ra + 3, 1)

        return carry

    lax.fori_loop(0, RPW // 2, pair, 0)


def _sc_pool(tabs, roist):
    mesh = plsc.VectorSubcoreMesh(core_axis_name="c", subcore_axis_name="s")
    return pl.kernel(
        _sc_body,
        mesh=mesh,
        compiler_params=pltpu.CompilerParams(needs_layout_passes=False),
        out_type=jax.ShapeDtypeStruct((NROI, C * NBIN), jnp.float32),
        scratch_types=[
            pltpu.VMEM((4, RPW), jnp.float32),
            pltpu.VMEM((RPW * ROISEG,), jnp.int32),
            pltpu.VMEM((2, ROISEG, C), jnp.float32),
            pltpu.VMEM((C * NBIN,), jnp.float32),
            pltpu.SemaphoreType.DMA,
            pltpu.SemaphoreType.DMA,
        ],
    )(tabs, roist)


@jax.jit
def kernel(FM, rois):
    FMt = jnp.transpose(FM, (1, 2, 0))
    tabs = _build_tables(FMt).reshape(NTAB * HW, C)
    roist = jnp.transpose(rois, (1, 0))
    out = _sc_pool(tabs, roist)
    return out.reshape(NROI, C, P, P)
